# Initial kernel scaffold; baseline (speedup 1.0000x reference)
import jax
import jax.numpy as jnp
from jax.experimental import pallas as pl


def kernel(x, emb, W, b):
    raise NotImplementedError("write your pallas kernel here")


# speedup vs baseline: 24.5160x; 24.5160x over previous
import functools

import jax
import jax.numpy as jnp
from jax import lax
from jax.experimental import pallas as pl
from jax.experimental.pallas import tpu as pltpu
from jax.experimental.pallas import tpu_sc as plsc

VOCAB = 100000
EMBED_DIM = 64
BATCH = 4096
HIST = 200

NUM_CORES = 2
NUM_SUBCORES = 16
LANES = 16
NUM_WORKERS = NUM_CORES * NUM_SUBCORES
ROWS_PER_W = BATCH // NUM_WORKERS
IDX_PER_W = ROWS_PER_W * HIST
GROUPS = ROWS_PER_W // LANES

_S_BLOCK = 4096
_S_GRID = (VOCAB + _S_BLOCK - 1) // _S_BLOCK
S_PAD = _S_GRID * _S_BLOCK


def _s_body(emb_ref, w_ref, b_ref, out_ref):
    e = emb_ref[...]
    w = w_ref[...]
    s = jax.lax.dot_general(
        w, e, (((1,), (1,)), ((), ())),
        preferred_element_type=jnp.float32)
    out_ref[...] = (s + b_ref[0]) * (1.0 / HIST)


def _compute_s(emb, W, b):
    return pl.pallas_call(
        _s_body,
        grid=(_S_GRID,),
        in_specs=[
            pl.BlockSpec((_S_BLOCK, EMBED_DIM), lambda i: (i, 0)),
            pl.BlockSpec((1, EMBED_DIM), lambda i: (0, 0)),
            pl.BlockSpec(memory_space=pltpu.SMEM),
        ],
        out_specs=pl.BlockSpec((1, _S_BLOCK), lambda i: (0, i)),
        out_shape=jax.ShapeDtypeStruct((1, S_PAD), jnp.float32),
    )(emb, W, b)


def _make_sc_kernel(interpret=False):
    mesh = plsc.VectorSubcoreMesh(
        core_axis_name="c", subcore_axis_name="s",
        num_cores=NUM_CORES, num_subcores=NUM_SUBCORES)

    @functools.partial(
        pl.kernel,
        mesh=mesh,
        out_type=jax.ShapeDtypeStruct((BATCH,), jnp.float32),
        scratch_types=[
            pltpu.VMEM((S_PAD,), jnp.float32),
            pltpu.VMEM((IDX_PER_W,), jnp.int32),
            pltpu.VMEM((ROWS_PER_W,), jnp.float32),
        ],
        compiler_params=pltpu.CompilerParams(needs_layout_passes=False),
        interpret=interpret,
    )
    def sc_kernel(s_hbm, xflat_hbm, out_hbm, table_v, idx_v, out_v):
        wid = lax.axis_index("s") * NUM_CORES + lax.axis_index("c")
        base = wid * IDX_PER_W
        pltpu.sync_copy(s_hbm, table_v)
        pltpu.sync_copy(xflat_hbm.at[pl.ds(base, IDX_PER_W)], idx_v)
        lane = lax.iota(jnp.int32, LANES)
        for g in range(GROUPS):
            base_vec = lane * HIST + (g * LANES * HIST)

            def body(l, acc, base_vec=base_vec):
                ind = plsc.load_gather(idx_v, [base_vec + l])
                vals = plsc.load_gather(table_v, [ind])
                return acc + vals

            acc = lax.fori_loop(0, HIST, body, jnp.zeros((LANES,), jnp.float32))
            out_v[pl.ds(g * LANES, LANES)] = 1.0 / (1.0 + jnp.exp(-acc))
        pltpu.sync_copy(
            out_v, out_hbm.at[pl.ds(wid * ROWS_PER_W, ROWS_PER_W)])

    return sc_kernel


_sc_kernel_cache = {}


def _get_sc_kernel():
    if "k" not in _sc_kernel_cache:
        _sc_kernel_cache["k"] = _make_sc_kernel()
    return _sc_kernel_cache["k"]


def kernel(x, emb, W, b):
    s = _compute_s(emb, W, b).reshape(S_PAD)
    xflat = x.reshape(-1).astype(jnp.int32)
    out = _get_sc_kernel()(s, xflat)
    return out.reshape(BATCH, 1)
